# Initial kernel scaffold; baseline (speedup 1.0000x reference)
import jax
import jax.numpy as jnp
from jax.experimental import pallas as pl


def kernel(features, tables, fm_W, fm_b, W1, b1, W2, b2, W3, b3, out_W, out_b):
    raise NotImplementedError("write your pallas kernel here")


# speedup vs baseline: 2.1746x; 2.1746x over previous
import functools

import jax
import jax.numpy as jnp
from jax import lax
from jax.experimental import pallas as pl
from jax.experimental.pallas import tpu as pltpu
from jax.experimental.pallas import tpu_sc as plsc

N_FIELDS = 26
VOCAB = 100000
EMB = 16
HIDDEN = [256, 128, 64]
BATCH = 16384

_D_IN = N_FIELDS * EMB
_ROWS = BATCH * N_FIELDS

_NC = 2
_NS = 16
_NW = _NC * _NS
_RPW = _ROWS // _NW
_CH = 1664
_NCHUNK = _RPW // _CH


def _sc_gather_body(tab_hbm, feat_hbm, out_hbm, feat_v, idx_v, rows_v, sem):
    wid = lax.axis_index("s") * _NC + lax.axis_index("c")
    base = wid * _RPW
    pltpu.sync_copy(feat_hbm.at[pl.ds(base, _RPW)], feat_v)
    lane = lax.iota(jnp.int32, 16)

    for c in range(_NCHUNK):
        start = base + c * _CH

        def idx_body(i, _, c=c):
            pos = base + c * _CH + i * 16 + lane
            field = lax.rem(pos, N_FIELDS)
            idx_v[pl.ds(i * 16, 16)] = (
                feat_v[pl.ds(c * _CH + i * 16, 16)] + field * VOCAB)
            return 0

        lax.fori_loop(0, _CH // 16, idx_body, 0)
        pltpu.async_copy(tab_hbm.at[idx_v], rows_v, sem).wait()
        pltpu.sync_copy(rows_v, out_hbm.at[pl.ds(start, _CH)])


def _sc_gather(tab_flat, feat_flat):
    mesh = plsc.VectorSubcoreMesh(
        core_axis_name="c", subcore_axis_name="s", num_cores=_NC)
    return pl.kernel(
        _sc_gather_body,
        out_type=jax.ShapeDtypeStruct((_ROWS, EMB), jnp.float32),
        mesh=mesh,
        scratch_types=[
            pltpu.VMEM((_RPW,), jnp.int32),
            pltpu.VMEM((_CH,), jnp.int32),
            pltpu.VMEM((_CH, EMB), jnp.float32),
            pltpu.SemaphoreType.DMA,
        ],
        compiler_params=pltpu.CompilerParams(use_tc_tiling_on_sc=False),
    )(tab_flat, feat_flat)


_BB = 1024


def _dense_body(x_ref, w1_ref, b1_ref, w2_ref, b2_ref, w3_ref, b3_ref,
                wfm_ref, wh_ref, c0_ref, out_ref):
    x = x_ref[...]
    h = jnp.maximum(
        jnp.dot(x, w1_ref[...], preferred_element_type=jnp.float32)
        + b1_ref[...], 0.0)
    h = jnp.maximum(
        jnp.dot(h, w2_ref[...], preferred_element_type=jnp.float32)
        + b2_ref[...], 0.0)
    h = jnp.maximum(
        jnp.dot(h, w3_ref[...], preferred_element_type=jnp.float32)
        + b3_ref[...], 0.0)
    fm = jnp.sum(x * wfm_ref[...], axis=1, keepdims=True)
    hs = jnp.sum(h * wh_ref[...], axis=1, keepdims=True)
    out_ref[...] = fm + hs + c0_ref[...]


def _dense(x, W1, b1, W2, b2, W3, b3, wfm, wh, c0):
    nblk = BATCH // _BB
    full = lambda a: pl.BlockSpec(a.shape, lambda i: (0,) * a.ndim)
    return pl.pallas_call(
        _dense_body,
        grid=(nblk,),
        in_specs=[
            pl.BlockSpec((_BB, _D_IN), lambda i: (i, 0)),
            full(W1), full(b1), full(W2), full(b2), full(W3), full(b3),
            full(wfm), full(wh), full(c0),
        ],
        out_specs=pl.BlockSpec((_BB, 1), lambda i: (i, 0)),
        out_shape=jax.ShapeDtypeStruct((BATCH, 1), jnp.float32),
    )(x, W1, b1, W2, b2, W3, b3, wfm, wh, c0)


def kernel(features, tables, fm_W, fm_b, W1, b1, W2, b2, W3, b3, out_W, out_b):
    feat_flat = features.astype(jnp.int32).reshape(-1)
    tab_flat = tables.reshape(N_FIELDS * VOCAB, EMB)
    x = _sc_gather(tab_flat, feat_flat).reshape(BATCH, _D_IN)

    a0 = out_W[0, 0]
    wfm = (fm_W[:, 0] * a0).reshape(1, _D_IN)
    wh = out_W[1:, 0].reshape(1, HIDDEN[2])
    c0 = (fm_b[0] * a0 + out_b[0]).reshape(1, 1)
    out = _dense(x, W1, b1.reshape(1, -1), W2, b2.reshape(1, -1),
                 W3, b3.reshape(1, -1), wfm, wh, c0)
    return out[:, 0]


# speedup vs baseline: 2.1899x; 1.0071x over previous
import functools

import jax
import jax.numpy as jnp
from jax import lax
from jax.experimental import pallas as pl
from jax.experimental.pallas import tpu as pltpu
from jax.experimental.pallas import tpu_sc as plsc

N_FIELDS = 26
VOCAB = 100000
EMB = 16
HIDDEN = [256, 128, 64]
BATCH = 16384

_D_IN = N_FIELDS * EMB
_ROWS = BATCH * N_FIELDS

_NC = 2
_NS = 16
_NW = _NC * _NS
_RPW = _ROWS // _NW
_CH = 1664
_NCHUNK = _RPW // _CH


def _sc_gather_body(tab_hbm, feat_hbm, out_hbm, feat_v, idx_a, idx_b,
                    rows_a, rows_b, sem_a, sem_b):
    wid = lax.axis_index("s") * _NC + lax.axis_index("c")
    base = wid * _RPW
    pltpu.sync_copy(feat_hbm.at[pl.ds(base, _RPW)], feat_v)
    lane = lax.iota(jnp.int32, 16)

    idxs = (idx_a, idx_b)
    bufs = (rows_a, rows_b)
    sems = (sem_a, sem_b)
    cps = []
    for c in range(_NCHUNK + 1):
        if c < _NCHUNK:
            idx_v = idxs[c % 2]

            def idx_body(i, _, c=c, idx_v=idx_v):
                pos = base + c * _CH + i * 16 + lane
                field = lax.rem(pos, N_FIELDS)
                idx_v[pl.ds(i * 16, 16)] = (
                    feat_v[pl.ds(c * _CH + i * 16, 16)] + field * VOCAB)
                return 0

            lax.fori_loop(0, _CH // 16, idx_body, 0)
            cps.append(
                pltpu.async_copy(tab_hbm.at[idx_v], bufs[c % 2], sems[c % 2]))
        if c > 0:
            cps[c - 1].wait()
            pltpu.sync_copy(bufs[(c - 1) % 2],
                            out_hbm.at[pl.ds(base + (c - 1) * _CH, _CH)])


def _sc_gather(tab_flat, feat_flat):
    mesh = plsc.VectorSubcoreMesh(
        core_axis_name="c", subcore_axis_name="s", num_cores=_NC)
    return pl.kernel(
        _sc_gather_body,
        out_type=jax.ShapeDtypeStruct((_ROWS, EMB), jnp.float32),
        mesh=mesh,
        scratch_types=[
            pltpu.VMEM((_RPW,), jnp.int32),
            pltpu.VMEM((_CH,), jnp.int32),
            pltpu.VMEM((_CH,), jnp.int32),
            pltpu.VMEM((_CH, EMB), jnp.float32),
            pltpu.VMEM((_CH, EMB), jnp.float32),
            pltpu.SemaphoreType.DMA,
            pltpu.SemaphoreType.DMA,
        ],
        compiler_params=pltpu.CompilerParams(use_tc_tiling_on_sc=False),
    )(tab_flat, feat_flat)


_BB = 1024


def _dense_body(x_ref, w1_ref, b1_ref, w2_ref, b2_ref, w3_ref, b3_ref,
                wfm_ref, wh_ref, c0_ref, out_ref):
    x = x_ref[...]
    h = jnp.maximum(
        jnp.dot(x, w1_ref[...], preferred_element_type=jnp.float32)
        + b1_ref[...], 0.0)
    h = jnp.maximum(
        jnp.dot(h, w2_ref[...], preferred_element_type=jnp.float32)
        + b2_ref[...], 0.0)
    h = jnp.maximum(
        jnp.dot(h, w3_ref[...], preferred_element_type=jnp.float32)
        + b3_ref[...], 0.0)
    fm = jnp.sum(x * wfm_ref[...], axis=1, keepdims=True)
    hs = jnp.sum(h * wh_ref[...], axis=1, keepdims=True)
    out_ref[...] = fm + hs + c0_ref[...]


def _dense(x, W1, b1, W2, b2, W3, b3, wfm, wh, c0):
    nblk = BATCH // _BB
    full = lambda a: pl.BlockSpec(a.shape, lambda i: (0,) * a.ndim)
    return pl.pallas_call(
        _dense_body,
        grid=(nblk,),
        in_specs=[
            pl.BlockSpec((_BB, _D_IN), lambda i: (i, 0)),
            full(W1), full(b1), full(W2), full(b2), full(W3), full(b3),
            full(wfm), full(wh), full(c0),
        ],
        out_specs=pl.BlockSpec((_BB, 1), lambda i: (i, 0)),
        out_shape=jax.ShapeDtypeStruct((BATCH, 1), jnp.float32),
    )(x, W1, b1, W2, b2, W3, b3, wfm, wh, c0)


def kernel(features, tables, fm_W, fm_b, W1, b1, W2, b2, W3, b3, out_W, out_b):
    feat_flat = features.astype(jnp.int32).reshape(-1)
    tab_flat = tables.reshape(N_FIELDS * VOCAB, EMB)
    x = _sc_gather(tab_flat, feat_flat).reshape(BATCH, _D_IN)

    a0 = out_W[0, 0]
    wfm = (fm_W[:, 0] * a0).reshape(1, _D_IN)
    wh = out_W[1:, 0].reshape(1, HIDDEN[2])
    c0 = (fm_b[0] * a0 + out_b[0]).reshape(1, 1)
    out = _dense(x, W1, b1.reshape(1, -1), W2, b2.reshape(1, -1),
                 W3, b3.reshape(1, -1), wfm, wh, c0)
    return out[:, 0]
